# baseline (speedup 1.0000x reference)
import functools

import jax
import jax.numpy as jnp
from jax import lax
from jax.experimental import pallas as pl
from jax.experimental.pallas import tpu as pltpu
from jax.experimental.pallas import tpu_sc as plsc

D = 64
L = 128
NW = 32
NBLK = 6400
PER_W = NBLK // NW


def _body(pair_hbm, acol_hbm, tab_hbm, out_hbm, pair_v, acol_v, gbuf, tbuf,
          sem_i, sem_g, sem_o):
    wid = lax.axis_index("s") * 2 + lax.axis_index("c")
    base = wid * PER_W

    def idx_copies(b, slot):
        m = base + b
        return (pltpu.make_async_copy(pair_hbm.at[m], pair_v.at[slot], sem_i),
                pltpu.make_async_copy(acol_hbm.at[m], acol_v.at[slot], sem_i))

    def out_copy(b, slot):
        m = base + b
        s8 = (m >> 7) * 8
        col = m & 127
        return pltpu.make_async_copy(
            tbuf.at[slot], out_hbm.at[pl.ds(s8, 8), pl.ds(col, 1)], sem_o)

    for b in (0, 1):
        for cp in idx_copies(b, b):
            cp.start()
    for b in (0, 1):
        for cp in idx_copies(b, b):
            cp.wait()
    pltpu.async_copy(tab_hbm.at[pair_v.at[0]], gbuf.at[0], sem_g)

    def step(b, slot):

        @pl.when(jnp.logical_and(b >= 1, b + 1 < PER_W))
        def _():
            for cp in idx_copies(b + 1, 1 - slot):
                cp.wait()

        pltpu.make_async_copy(
            tab_hbm.at[pair_v.at[slot]], gbuf.at[slot], sem_g).wait()

        @pl.when(b + 1 < PER_W)
        def _():
            pltpu.async_copy(
                tab_hbm.at[pair_v.at[1 - slot]], gbuf.at[1 - slot], sem_g)

        @pl.when(b >= 2)
        def _():
            out_copy(b - 2, slot).wait()

        def tr_body(tr, carry):
            for g in range(8):
                rows = lax.iota(jnp.int32, 16) + (16 * g)
                cols0 = acol_v[slot, pl.ds(16 * g, 16)]
                for r in range(8):
                    cols = cols0 + (tr * 8 + r)
                    v = plsc.load_gather(gbuf.at[slot], [rows, cols])
                    tbuf[slot, tr, 0, r, pl.ds(16 * g, 16)] = v
            return carry

        lax.fori_loop(0, 8, tr_body, 0, unroll=False)

        out_copy(b, slot).start()

        @pl.when(b + 2 < PER_W)
        def _():
            for cp in idx_copies(b + 2, slot):
                cp.start()

    def pair_step(p, carry):
        step(p * 2, 0)
        step(p * 2 + 1, 1)
        return carry

    lax.fori_loop(0, PER_W // 2, pair_step, 0, unroll=False)

    for tail in (PER_W - 2, PER_W - 1):
        out_copy(tail, tail % 2).wait()


def kernel(idx, weight):
    B, S = idx.shape
    idxT = idx.T.astype(jnp.int32)
    pair = (idxT >> 1).reshape(NBLK, L)
    acol = ((idxT & 1) << 6).reshape(NBLK, L)
    tab = weight.reshape(weight.shape[0] // 2, 2 * D)

    grid_kernel = functools.partial(
        pl.kernel,
        out_type=jax.ShapeDtypeStruct((S * 8, L, 8, L), jnp.float32),
        mesh=plsc.VectorSubcoreMesh(core_axis_name="c", subcore_axis_name="s"),
        scratch_types=[
            pltpu.VMEM((2, L), jnp.int32),
            pltpu.VMEM((2, L), jnp.int32),
            pltpu.VMEM((2, L, 2 * D), jnp.float32),
            pltpu.VMEM((2, 8, 1, 8, L), jnp.float32),
            pltpu.SemaphoreType.DMA,
            pltpu.SemaphoreType.DMA,
            pltpu.SemaphoreType.DMA,
        ],
        compiler_params=pltpu.CompilerParams(needs_layout_passes=False),
    )
    out4 = grid_kernel(_body)(pair, acol, tab)
    out5 = out4.reshape(S, 8, L, 8, L)
    return out5.transpose(2, 4, 0, 1, 3).reshape(B, S, D)


# speedup vs baseline: 1.1881x; 1.1881x over previous
import functools

import jax
import jax.numpy as jnp
from jax import lax
from jax.experimental import pallas as pl
from jax.experimental.pallas import tpu as pltpu
from jax.experimental.pallas import tpu_sc as plsc

D = 64
L = 128
NW = 32
NBLK = 6400
PER_W = NBLK // NW


def _body(pair_hbm, acol_hbm, tab_hbm, out_hbm, pair_v, acol_v, gbuf, tbuf,
          sem_i, sem_g, sem_o):
    wid = lax.axis_index("s") * 2 + lax.axis_index("c")
    base = wid * PER_W

    def idx_copies(b, slot):
        m = base + b
        return (pltpu.make_async_copy(pair_hbm.at[m], pair_v.at[slot], sem_i),
                pltpu.make_async_copy(acol_hbm.at[m], acol_v.at[slot], sem_i))

    def out_copy(b, slot):
        m = base + b
        s8 = (m >> 7) * 8
        col = m & 127
        return pltpu.make_async_copy(
            tbuf.at[slot], out_hbm.at[pl.ds(s8, 8), pl.ds(col, 1)], sem_o)

    for b in (0, 1):
        for cp in idx_copies(b, b):
            cp.start()
    for b in (0, 1):
        for cp in idx_copies(b, b):
            cp.wait()
    pltpu.async_copy(
        tab_hbm.at[pair_v.at[0]], gbuf.at[0, :, pl.ds(0, 2 * D)], sem_g)

    def step(b, slot):

        @pl.when(jnp.logical_and(b >= 1, b + 1 < PER_W))
        def _():
            for cp in idx_copies(b + 1, 1 - slot):
                cp.wait()

        pltpu.make_async_copy(
            tab_hbm.at[pair_v.at[slot]],
            gbuf.at[slot, :, pl.ds(0, 2 * D)], sem_g).wait()

        @pl.when(b + 1 < PER_W)
        def _():
            pltpu.async_copy(
                tab_hbm.at[pair_v.at[1 - slot]],
                gbuf.at[1 - slot, :, pl.ds(0, 2 * D)], sem_g)

        @pl.when(b >= 2)
        def _():
            out_copy(b - 2, slot).wait()

        rows_g = [lax.iota(jnp.int32, 16) + (16 * g) for g in range(8)]
        acol_g = [acol_v[slot, pl.ds(16 * g, 16)] for g in range(8)]
        for tr in range(8):
            for r in range(8):
                d = tr * 8 + r
                vals = [
                    plsc.load_gather(gbuf.at[slot],
                                     [rows_g[g], acol_g[g] + d])
                    for g in range(8)
                ]
                for g in range(8):
                    tbuf[slot, tr, 0, r, pl.ds(16 * g, 16)] = vals[g]

        out_copy(b, slot).start()

        @pl.when(b + 2 < PER_W)
        def _():
            for cp in idx_copies(b + 2, slot):
                cp.start()

    def pair_step(p, carry):
        step(p * 2, 0)
        step(p * 2 + 1, 1)
        return carry

    lax.fori_loop(0, PER_W // 2, pair_step, 0, unroll=False)

    for tail in (PER_W - 2, PER_W - 1):
        out_copy(tail, tail % 2).wait()


def kernel(idx, weight):
    B, S = idx.shape
    idxT = idx.T.astype(jnp.int32)
    pair = (idxT >> 1).reshape(NBLK, L)
    acol = ((idxT & 1) << 6).reshape(NBLK, L)
    tab = weight.reshape(weight.shape[0] // 2, 2 * D)

    grid_kernel = functools.partial(
        pl.kernel,
        out_type=jax.ShapeDtypeStruct((S * 8, L, 8, L), jnp.float32),
        mesh=plsc.VectorSubcoreMesh(core_axis_name="c", subcore_axis_name="s"),
        scratch_types=[
            pltpu.VMEM((2, L), jnp.int32),
            pltpu.VMEM((2, L), jnp.int32),
            pltpu.VMEM((2, L, 2 * D + 1), jnp.float32),
            pltpu.VMEM((2, 8, 1, 8, L), jnp.float32),
            pltpu.SemaphoreType.DMA,
            pltpu.SemaphoreType.DMA,
            pltpu.SemaphoreType.DMA,
        ],
        compiler_params=pltpu.CompilerParams(needs_layout_passes=False),
    )
    out4 = grid_kernel(_body)(pair, acol, tab)
    out5 = out4.reshape(S, 8, L, 8, L)
    return out5.transpose(2, 4, 0, 1, 3).reshape(B, S, D)


# speedup vs baseline: 1.9105x; 1.6079x over previous
import functools

import jax
import jax.numpy as jnp
from jax import lax
from jax.experimental import pallas as pl
from jax.experimental.pallas import tpu as pltpu
from jax.experimental.pallas import tpu_sc as plsc

D = 64
L = 128
NW = 32
NBLK = 6400
PER_W = NBLK // NW
NS = 4


def _body(idx_hbm, tab_hbm, out_hbm, idx_v, gbuf, sem_i, sem_g, sem_o):
    wid = lax.axis_index("s") * 2 + lax.axis_index("c")
    base = wid * PER_W

    def idx_copy(b, slot):
        return pltpu.make_async_copy(
            idx_hbm.at[base + b], idx_v.at[slot], sem_i)

    def gather_copy(slot):
        return pltpu.make_async_copy(
            tab_hbm.at[idx_v.at[slot]], gbuf.at[slot], sem_g)

    def out_copy(b, slot):
        m = base + b
        row = (m >> 7) * 16384 + (m & 127) * 128
        return pltpu.make_async_copy(
            gbuf.at[slot], out_hbm.at[pl.ds(row, L)], sem_o)

    for b in range(NS):
        idx_copy(b, b).start()
    for b in range(NS):
        idx_copy(b, b).wait()
    gather_copy(0).start()

    def step(b, slot):

        @pl.when(jnp.logical_and(b >= NS - 1, b + 1 < PER_W))
        def _():
            idx_copy(b + 1, (b + 1) % NS).wait()

        gather_copy(slot).wait()

        out_copy(b, slot).start()

        @pl.when(b >= NS - 1)
        def _():
            out_copy(b - (NS - 1), (b + 1) % NS).wait()

        @pl.when(b + 1 < PER_W)
        def _():
            gather_copy((b + 1) % NS).start()

        @pl.when(b + NS < PER_W)
        def _():
            idx_copy(b + NS, slot).start()

    def quad_step(q, carry):
        for s in range(NS):
            step(q * NS + s, s)
        return carry

    lax.fori_loop(0, PER_W // NS, quad_step, 0, unroll=False)

    for tail in range(PER_W - (NS - 1), PER_W):
        out_copy(tail, tail % NS).wait()


def kernel(idx, weight):
    B, S = idx.shape
    V = weight.shape[0]
    idxT = idx.T.astype(jnp.int32).reshape(NBLK, L)
    tab = jnp.pad(weight, ((0, 0), (0, 2 * D - D)))

    grid_kernel = functools.partial(
        pl.kernel,
        out_type=jax.ShapeDtypeStruct((S * B, 2 * D), jnp.float32),
        mesh=plsc.VectorSubcoreMesh(core_axis_name="c", subcore_axis_name="s"),
        scratch_types=[
            pltpu.VMEM((NS, L), jnp.int32),
            pltpu.VMEM((NS, L, 2 * D), jnp.float32),
            pltpu.SemaphoreType.DMA,
            pltpu.SemaphoreType.DMA,
            pltpu.SemaphoreType.DMA,
        ],
    )
    out3 = grid_kernel(_body)(idxT, tab)
    o = out3.reshape(S, B, 2 * D)[:, :, :D]
    return o.transpose(1, 0, 2)


# speedup vs baseline: 1.9168x; 1.0033x over previous
import functools

import jax
import jax.numpy as jnp
from jax import lax
from jax.experimental import pallas as pl
from jax.experimental.pallas import tpu as pltpu
from jax.experimental.pallas import tpu_sc as plsc

D = 64
L = 128
NW = 32
NBLK = 6400
PER_W = NBLK // NW
NS = 5


def _body(idx_hbm, tab_hbm, out_hbm, idx_v, gbuf, sem_i, sem_g, sem_o):
    wid = lax.axis_index("s") * 2 + lax.axis_index("c")
    base = wid * PER_W

    def idx_copy(b, slot):
        return pltpu.make_async_copy(
            idx_hbm.at[base + b], idx_v.at[slot], sem_i)

    def gather_copy(slot):
        return pltpu.make_async_copy(
            tab_hbm.at[idx_v.at[slot]], gbuf.at[slot], sem_g)

    def out_copy(b, slot):
        m = base + b
        row = (m >> 7) * 16384 + (m & 127) * 128
        return pltpu.make_async_copy(
            gbuf.at[slot], out_hbm.at[pl.ds(row, L)], sem_o)

    for b in range(NS):
        idx_copy(b, b).start()
    for b in range(NS):
        idx_copy(b, b).wait()
    gather_copy(0).start()

    def step(b, slot):

        @pl.when(jnp.logical_and(b >= NS - 1, b + 1 < PER_W))
        def _():
            idx_copy(b + 1, (b + 1) % NS).wait()

        gather_copy(slot).wait()

        out_copy(b, slot).start()

        @pl.when(b >= NS - 1)
        def _():
            out_copy(b - (NS - 1), (b + 1) % NS).wait()

        @pl.when(b + 1 < PER_W)
        def _():
            gather_copy((b + 1) % NS).start()

        @pl.when(b + NS < PER_W)
        def _():
            idx_copy(b + NS, slot).start()

    def quad_step(q, carry):
        for s in range(NS):
            step(q * NS + s, s)
        return carry

    lax.fori_loop(0, PER_W // NS, quad_step, 0, unroll=False)

    for tail in range(PER_W - (NS - 1), PER_W):
        out_copy(tail, tail % NS).wait()


def kernel(idx, weight):
    B, S = idx.shape
    V = weight.shape[0]
    idxT = idx.T.astype(jnp.int32).reshape(NBLK, L)
    tab = jnp.pad(weight, ((0, 0), (0, D)))

    grid_kernel = functools.partial(
        pl.kernel,
        out_type=jax.ShapeDtypeStruct((S * B, 2 * D), jnp.float32),
        mesh=plsc.VectorSubcoreMesh(core_axis_name="c", subcore_axis_name="s"),
        scratch_types=[
            pltpu.VMEM((NS, L), jnp.int32),
            pltpu.VMEM((NS, L, 2 * D), jnp.float32),
            pltpu.SemaphoreType.DMA,
            pltpu.SemaphoreType.DMA,
            pltpu.SemaphoreType.DMA,
        ],
    )
    out3 = grid_kernel(_body)(idxT, tab)
    o = out3.reshape(S, B, 2 * D)[:, :, :D]
    return o.transpose(1, 0, 2)


# speedup vs baseline: 1.9184x; 1.0009x over previous
import functools

import jax
import jax.numpy as jnp
from jax import lax
from jax.experimental import pallas as pl
from jax.experimental.pallas import tpu as pltpu
from jax.experimental.pallas import tpu_sc as plsc

D = 64
L = 128
NW = 32
NBLK = 6400
PER_W = NBLK // NW
NS = 5


def _body(idx_hbm, tab_hbm, out_hbm, idx_v, gbuf, sem_i, sem_g, sem_o):
    wid = lax.axis_index("s") * 2 + lax.axis_index("c")
    base = wid * PER_W

    def idx_copy(b, slot):
        return pltpu.make_async_copy(
            idx_hbm.at[base + b], idx_v.at[slot], sem_i)

    def gather_copy(slot):
        return pltpu.make_async_copy(
            tab_hbm.at[idx_v.at[slot]], gbuf.at[slot], sem_g)

    def out_copy(b, slot):
        m = base + b
        row = (m >> 7) * 16384 + (m & 127) * 128
        return pltpu.make_async_copy(
            gbuf.at[slot], out_hbm.at[pl.ds(row, L)], sem_o)

    for b in range(NS):
        idx_copy(b, b).start()
    for b in range(NS):
        idx_copy(b, b).wait()
    gather_copy(0).start()

    def step(b, slot):

        @pl.when(jnp.logical_and(b >= NS - 1, b + 1 < PER_W))
        def _():
            idx_copy(b + 1, (b + 1) % NS).wait()

        gather_copy(slot).wait()

        out_copy(b, slot).start()

        @pl.when(b >= NS - 1)
        def _():
            out_copy(b - (NS - 1), (b + 1) % NS).wait()

        @pl.when(b + 1 < PER_W)
        def _():
            gather_copy((b + 1) % NS).start()

        @pl.when(b + NS < PER_W)
        def _():
            idx_copy(b + NS, slot).start()

    def group_step(q, carry):
        for s in range(NS):
            step(q * NS + s, s)
        return carry

    lax.fori_loop(0, PER_W // NS, group_step, 0, unroll=False)

    for tail in range(PER_W - (NS - 1), PER_W):
        out_copy(tail, tail % NS).wait()


def kernel(idx, weight):
    B, S = idx.shape
    V = weight.shape[0]
    idxT = idx.T.astype(jnp.int32).reshape(NBLK, L)
    tab = jnp.pad(weight, ((0, 0), (0, D)))

    grid_kernel = functools.partial(
        pl.kernel,
        out_type=jax.ShapeDtypeStruct((S * B, 2 * D), jnp.float32),
        mesh=plsc.VectorSubcoreMesh(core_axis_name="c", subcore_axis_name="s"),
        scratch_types=[
            pltpu.VMEM((NS, L), jnp.int32),
            pltpu.VMEM((NS, L, 2 * D), jnp.float32),
            pltpu.SemaphoreType.DMA,
            pltpu.SemaphoreType.DMA,
            pltpu.SemaphoreType.DMA,
        ],
    )
    out3 = grid_kernel(_body)(idxT, tab)
    o = out3.reshape(S, B, 2 * D)[:, :, :D]
    return o.transpose(1, 0, 2)
